# Initial kernel scaffold; baseline (speedup 1.0000x reference)
import jax
import jax.numpy as jnp
from jax.experimental import pallas as pl


def kernel(z, W):
    raise NotImplementedError("write your pallas kernel here")


# speedup vs baseline: 1.4681x; 1.4681x over previous
import functools

import jax
import jax.numpy as jnp
from jax import lax
from jax.experimental import pallas as pl
from jax.experimental.pallas import tpu as pltpu
from jax.experimental.pallas import tpu_sc as plsc

B, D, T, K = 16, 256, 2048, 8192
TT = 512
WINDOWS = (2736, 5472, 8192)
NT = T // TT


def _vq_body(z_ref, x2_ref, w_ref, idx_ref, loss_ref):
    nt = pl.program_id(1)
    zb = z_ref[0]
    x2 = x2_ref[0, 0, :]

    carry_val = None
    carry_idx = None
    carry_true = None
    lo = 0
    for hi in WINDOWS:
        kt = hi - lo
        wc = w_ref[lo:hi, :]
        y2 = jnp.sum(wc * wc, axis=1)
        zw = lax.dot_general(wc, zb,
                             dimension_numbers=(((1,), (0,)), ((), ())),
                             preferred_element_type=jnp.float32)
        dist = (x2[None, :] + y2[:, None]) - 2.0 * zw
        wmin = jnp.min(dist, axis=0)
        kio = lax.broadcasted_iota(jnp.int32, (kt, TT), 0) + lo
        widx = jnp.min(
            jnp.where(dist == wmin[None, :], kio, jnp.int32(K)), axis=0)
        if carry_val is None:
            carry_val, carry_idx, carry_true = wmin, widx, wmin
        else:
            take = wmin < carry_val
            carry_idx = jnp.where(take, widx, carry_idx)
            carry_true = jnp.where(take, wmin, carry_true)
            carry_val = jnp.where(take, wmin, carry_val)
        carry_val = carry_val.astype(jnp.bfloat16).astype(jnp.float32)
        lo = hi

    idx_ref[0, 0, pl.ds(nt * TT, TT)] = carry_idx

    part = jnp.sum(carry_true)
    lane = lax.broadcasted_iota(jnp.int32, (128,), 0)
    prev = jnp.where(nt == 0, jnp.zeros((128,), jnp.float32), loss_ref[0, 0, :])
    loss_ref[0, 0, :] = prev + jnp.where(lane == 0, part, 0.0)


def _vq_argmin(z, x2, w):
    return pl.pallas_call(
        _vq_body,
        grid=(B, NT),
        in_specs=[
            pl.BlockSpec((1, D, TT), lambda b, nt: (b, 0, nt)),
            pl.BlockSpec((1, 1, TT), lambda b, nt: (b, 0, nt)),
            pl.BlockSpec((K, D), lambda b, nt: (0, 0)),
        ],
        out_specs=[
            pl.BlockSpec((1, 1, T), lambda b, nt: (b, 0, 0)),
            pl.BlockSpec((1, 1, 128), lambda b, nt: (b, 0, 0)),
        ],
        out_shape=[
            jax.ShapeDtypeStruct((B, 1, T), jnp.int32),
            jax.ShapeDtypeStruct((B, 1, 128), jnp.float32),
        ],
        compiler_params=pltpu.CompilerParams(
            dimension_semantics=("parallel", "arbitrary")),
    )(z, x2, w)


N = B * T
NC, NS = 2, 16
NW = NC * NS
CHUNK = 128
CH = N // (NW * CHUNK)

_sc_mesh = plsc.VectorSubcoreMesh(core_axis_name="c", subcore_axis_name="s")


@functools.partial(
    pl.kernel,
    mesh=_sc_mesh,
    out_type=jax.ShapeDtypeStruct((N, D), jnp.float32),
    scratch_types=[
        pltpu.VMEM((CH, CHUNK), jnp.int32),
        pltpu.VMEM((CHUNK, D), jnp.float32),
        pltpu.VMEM((CHUNK, D), jnp.float32),
        pltpu.SemaphoreType.DMA,
        pltpu.SemaphoreType.DMA,
    ],
)
def _sc_gather(table_hbm, idx_hbm, out_hbm, idx_v, buf0, buf1, sem0, sem1):
    wid = lax.axis_index("s") * NC + lax.axis_index("c")
    base = wid * (CH * CHUNK)
    pltpu.sync_copy(idx_hbm.at[wid], idx_v)
    bufs = (buf0, buf1)
    sems = (sem0, sem1)
    copies = [None, None]
    copies[0] = pltpu.async_copy(table_hbm.at[idx_v.at[0]], bufs[0], sems[0])
    for c in range(CH):
        p = c % 2
        copies[p].wait()
        if c + 1 < CH:
            copies[1 - p] = pltpu.async_copy(
                table_hbm.at[idx_v.at[c + 1]], bufs[1 - p], sems[1 - p])
        pltpu.sync_copy(bufs[p], out_hbm.at[pl.ds(base + c * CHUNK, CHUNK)])


def kernel(z, W):
    zt = jnp.transpose(z, (0, 2, 1))
    z_flat = zt.reshape(-1, D)
    x2 = jnp.sum(z_flat ** 2, axis=1)
    idx3, loss_parts = _vq_argmin(z, x2.reshape(B, 1, T), W)
    codes = idx3.reshape(1, B, 1, T)
    loss_commit = jnp.sum(loss_parts) / jnp.float32(B * T * D)
    q = _sc_gather(W, idx3.reshape(NW, CH, CHUNK))
    z_q = q.reshape(B, T, D).transpose(0, 2, 1)
    loss_codebook = jnp.float32(0.0)
    bandwidth = jnp.array([0.075], dtype=jnp.float32)
    return (z_q, codes, loss_commit, loss_codebook, bandwidth)


# speedup vs baseline: 1.5594x; 1.0622x over previous
import functools

import jax
import jax.numpy as jnp
from jax import lax
from jax.experimental import pallas as pl
from jax.experimental.pallas import tpu as pltpu
from jax.experimental.pallas import tpu_sc as plsc

B, D, T, K = 16, 256, 2048, 8192
TT = 512
WINDOWS = (2736, 5472, 8192)
NT = T // TT


def _vq_body(z_ref, x2_ref, y2_ref, w2_ref, idx_ref, loss_ref):
    nt = pl.program_id(1)
    zb = z_ref[0]
    x2 = x2_ref[0, 0, :]

    carry_val = None
    carry_idx = None
    carry_true = None
    lo = 0
    for hi in WINDOWS:
        kt = hi - lo
        wc = w2_ref[lo:hi, :]
        y2 = y2_ref[lo:hi, :]
        zw2 = lax.dot_general(wc, zb,
                              dimension_numbers=(((1,), (0,)), ((), ())),
                              preferred_element_type=jnp.float32)
        dist = (x2[None, :] + y2) - zw2
        wmin = jnp.min(dist, axis=0)
        kio = lax.broadcasted_iota(jnp.int32, (kt, TT), 0) + lo
        widx = jnp.min(
            jnp.where(dist == wmin[None, :], kio, jnp.int32(K)), axis=0)
        if carry_val is None:
            carry_val, carry_idx, carry_true = wmin, widx, wmin
        else:
            take = wmin < carry_val
            carry_idx = jnp.where(take, widx, carry_idx)
            carry_true = jnp.where(take, wmin, carry_true)
            carry_val = jnp.where(take, wmin, carry_val)
        carry_val = carry_val.astype(jnp.bfloat16).astype(jnp.float32)
        lo = hi

    idx_ref[0, 0, pl.ds(nt * TT, TT)] = carry_idx

    part = jnp.sum(carry_true)
    lane = lax.broadcasted_iota(jnp.int32, (128,), 0)
    prev = jnp.where(nt == 0, jnp.zeros((128,), jnp.float32), loss_ref[0, 0, :])
    loss_ref[0, 0, :] = prev + jnp.where(lane == 0, part, 0.0)


def _vq_argmin(z, x2, y2, w2):
    return pl.pallas_call(
        _vq_body,
        grid=(B, NT),
        in_specs=[
            pl.BlockSpec((1, D, TT), lambda b, nt: (b, 0, nt)),
            pl.BlockSpec((1, 1, TT), lambda b, nt: (b, 0, nt)),
            pl.BlockSpec((K, 1), lambda b, nt: (0, 0)),
            pl.BlockSpec((K, D), lambda b, nt: (0, 0)),
        ],
        out_specs=[
            pl.BlockSpec((1, 1, T), lambda b, nt: (b, 0, 0)),
            pl.BlockSpec((1, 1, 128), lambda b, nt: (b, 0, 0)),
        ],
        out_shape=[
            jax.ShapeDtypeStruct((B, 1, T), jnp.int32),
            jax.ShapeDtypeStruct((B, 1, 128), jnp.float32),
        ],
        compiler_params=pltpu.CompilerParams(
            dimension_semantics=("parallel", "arbitrary")),
    )(z, x2, y2, w2)


N = B * T
NC, NS = 2, 16
NW = NC * NS
CHUNK = 128
CH = N // (NW * CHUNK)

@functools.cache
def _sc_gather_fn():
    mesh = plsc.VectorSubcoreMesh(core_axis_name="c", subcore_axis_name="s")

    @functools.partial(
        pl.kernel,
        mesh=mesh,
        out_type=jax.ShapeDtypeStruct((N, D), jnp.float32),
        scratch_types=[
            pltpu.VMEM((CH, CHUNK), jnp.int32),
            pltpu.VMEM((CHUNK, D), jnp.float32),
            pltpu.VMEM((CHUNK, D), jnp.float32),
            pltpu.SemaphoreType.DMA,
            pltpu.SemaphoreType.DMA,
        ],
    )
    def _sc_gather(table_hbm, idx_hbm, out_hbm, idx_v, buf0, buf1, sem0, sem1):
        wid = lax.axis_index("s") * NC + lax.axis_index("c")
        base = wid * (CH * CHUNK)
        pltpu.sync_copy(idx_hbm.at[wid], idx_v)
        bufs = (buf0, buf1)
        sems = (sem0, sem1)
        copies = [None, None]
        copies[0] = pltpu.async_copy(table_hbm.at[idx_v.at[0]], bufs[0], sems[0])
        for c in range(CH):
            p = c % 2
            copies[p].wait()
            if c + 1 < CH:
                copies[1 - p] = pltpu.async_copy(
                    table_hbm.at[idx_v.at[c + 1]], bufs[1 - p], sems[1 - p])
            pltpu.sync_copy(bufs[p], out_hbm.at[pl.ds(base + c * CHUNK, CHUNK)])

    return _sc_gather


def kernel(z, W):
    zt = jnp.transpose(z, (0, 2, 1))
    z_flat = zt.reshape(-1, D)
    x2 = jnp.sum(z_flat ** 2, axis=1)
    y2 = jnp.sum(W ** 2, axis=1)
    idx3, loss_parts = _vq_argmin(
        z, x2.reshape(B, 1, T), y2.reshape(K, 1), 2.0 * W)
    codes = idx3.reshape(1, B, 1, T)
    loss_commit = jnp.sum(loss_parts) / jnp.float32(B * T * D)
    q = _sc_gather_fn()(W, idx3.reshape(NW, CH, CHUNK))
    z_q = q.reshape(B, T, D).transpose(0, 2, 1)
    loss_codebook = jnp.float32(0.0)
    bandwidth = jnp.array([0.075], dtype=jnp.float32)
    return (z_q, codes, loss_commit, loss_codebook, bandwidth)


# speedup vs baseline: 1.5712x; 1.0075x over previous
import functools

import jax
import jax.numpy as jnp
from jax import lax
from jax.experimental import pallas as pl
from jax.experimental.pallas import tpu as pltpu
from jax.experimental.pallas import tpu_sc as plsc

B, D, T, K = 16, 256, 2048, 8192
TT = 512
WINDOWS = (2736, 5472, 8192)
NT = T // TT


def _vq_body(z_ref, x2_ref, y2_ref, w2_ref, idx_ref, loss_ref):
    nt = pl.program_id(1)
    zb = z_ref[0]
    x2 = x2_ref[0, 0, :]

    carry_val = None
    carry_idx = None
    carry_true = None
    lo = 0
    for hi in WINDOWS:
        kt = hi - lo
        wc = w2_ref[lo:hi, :]
        y2 = y2_ref[lo:hi, :]
        zw2 = lax.dot_general(wc, zb,
                              dimension_numbers=(((1,), (0,)), ((), ())),
                              preferred_element_type=jnp.float32)
        dist = (x2[None, :] + y2) - zw2
        wmin = jnp.min(dist, axis=0)
        kio = lax.broadcasted_iota(jnp.int32, (kt, TT), 0) + lo
        widx = jnp.min(
            jnp.where(dist == wmin[None, :], kio, jnp.int32(K)), axis=0)
        if carry_val is None:
            carry_val, carry_idx, carry_true = wmin, widx, wmin
        else:
            take = wmin < carry_val
            carry_idx = jnp.where(take, widx, carry_idx)
            carry_true = jnp.where(take, wmin, carry_true)
            carry_val = jnp.where(take, wmin, carry_val)
        carry_val = carry_val.astype(jnp.bfloat16).astype(jnp.float32)
        lo = hi

    idx_ref[0, 0, pl.ds(nt * TT, TT)] = carry_idx

    part = jnp.sum(carry_true)
    lane = lax.broadcasted_iota(jnp.int32, (128,), 0)
    prev = jnp.where(nt == 0, jnp.zeros((128,), jnp.float32), loss_ref[0, 0, :])
    loss_ref[0, 0, :] = prev + jnp.where(lane == 0, part, 0.0)


def _vq_argmin(z, x2, y2, w2):
    return pl.pallas_call(
        _vq_body,
        grid=(B, NT),
        in_specs=[
            pl.BlockSpec((1, D, TT), lambda b, nt: (b, 0, nt)),
            pl.BlockSpec((1, 1, TT), lambda b, nt: (b, 0, nt)),
            pl.BlockSpec((K, 1), lambda b, nt: (0, 0)),
            pl.BlockSpec((K, D), lambda b, nt: (0, 0)),
        ],
        out_specs=[
            pl.BlockSpec((1, 1, T), lambda b, nt: (b, 0, 0)),
            pl.BlockSpec((1, 1, 128), lambda b, nt: (b, 0, 0)),
        ],
        out_shape=[
            jax.ShapeDtypeStruct((B, 1, T), jnp.int32),
            jax.ShapeDtypeStruct((B, 1, 128), jnp.float32),
        ],
        compiler_params=pltpu.CompilerParams(
            dimension_semantics=("parallel", "arbitrary")),
    )(z, x2, y2, w2)


N = B * T
NC, NS = 2, 16
NW = NC * NS
CHUNK = 128
CH = N // (NW * CHUNK)

@functools.cache
def _sc_gather_fn():
    mesh = plsc.VectorSubcoreMesh(core_axis_name="c", subcore_axis_name="s")

    @functools.partial(
        pl.kernel,
        mesh=mesh,
        out_type=jax.ShapeDtypeStruct((N, D), jnp.float32),
        scratch_types=[
            pltpu.VMEM((CH, CHUNK), jnp.int32),
            pltpu.VMEM((CHUNK, D), jnp.float32),
            pltpu.VMEM((CHUNK, D), jnp.float32),
            pltpu.SemaphoreType.DMA,
            pltpu.SemaphoreType.DMA,
        ],
    )
    def _sc_gather(table_hbm, idx_hbm, out_hbm, idx_v, buf0, buf1, sem0, sem1):
        wid = lax.axis_index("s") * NC + lax.axis_index("c")
        base = wid * (CH * CHUNK)
        pltpu.sync_copy(idx_hbm.at[wid], idx_v)
        bufs = (buf0, buf1)
        sems = (sem0, sem1)
        copies = [None, None]
        copies[0] = pltpu.async_copy(table_hbm.at[idx_v.at[0]], bufs[0], sems[0])
        for c in range(CH):
            p = c % 2
            copies[p].wait()
            if c + 1 < CH:
                copies[1 - p] = pltpu.async_copy(
                    table_hbm.at[idx_v.at[c + 1]], bufs[1 - p], sems[1 - p])
            pltpu.sync_copy(bufs[p], out_hbm.at[pl.ds(base + c * CHUNK, CHUNK)])

    return _sc_gather


def kernel(z, W):
    x2 = jnp.sum(z * z, axis=1).reshape(-1)
    y2 = jnp.sum(W ** 2, axis=1)
    idx3, loss_parts = _vq_argmin(
        z, x2.reshape(B, 1, T), y2.reshape(K, 1), 2.0 * W)
    codes = idx3.reshape(1, B, 1, T)
    loss_commit = jnp.sum(loss_parts) / jnp.float32(B * T * D)
    q = _sc_gather_fn()(W, idx3.reshape(NW, CH, CHUNK))
    z_q = q.reshape(B, T, D).transpose(0, 2, 1)
    loss_codebook = jnp.float32(0.0)
    bandwidth = jnp.array([0.075], dtype=jnp.float32)
    return (z_q, codes, loss_commit, loss_codebook, bandwidth)


# speedup vs baseline: 1.6562x; 1.0541x over previous
import functools

import jax
import jax.numpy as jnp
from jax import lax
from jax.experimental import pallas as pl
from jax.experimental.pallas import tpu as pltpu
from jax.experimental.pallas import tpu_sc as plsc

B, D, T, K = 16, 256, 2048, 8192
TT = 1024
WINDOWS = (2736, 5472, 8192)
NT = T // TT


def _vq_body(z_ref, x2_ref, y2_ref, w2_ref, idx_ref, loss_ref):
    nt = pl.program_id(1)
    zb = z_ref[0]
    x2 = x2_ref[0, 0, :]

    carry_val = None
    carry_idx = None
    carry_true = None
    lo = 0
    for hi in WINDOWS:
        kt = hi - lo
        wc = w2_ref[lo:hi, :]
        y2 = y2_ref[lo:hi, :]
        zw2 = lax.dot_general(wc, zb,
                              dimension_numbers=(((1,), (0,)), ((), ())),
                              preferred_element_type=jnp.float32)
        dist = (x2[None, :] + y2) - zw2
        wmin = jnp.min(dist, axis=0)
        kio = lax.broadcasted_iota(jnp.int32, (kt, TT), 0) + lo
        widx = jnp.min(
            jnp.where(dist == wmin[None, :], kio, jnp.int32(K)), axis=0)
        if carry_val is None:
            carry_val, carry_idx, carry_true = wmin, widx, wmin
        else:
            take = wmin < carry_val
            carry_idx = jnp.where(take, widx, carry_idx)
            carry_true = jnp.where(take, wmin, carry_true)
            carry_val = jnp.where(take, wmin, carry_val)
        carry_val = carry_val.astype(jnp.bfloat16).astype(jnp.float32)
        lo = hi

    idx_ref[0, 0, pl.ds(nt * TT, TT)] = carry_idx

    part = jnp.sum(carry_true)
    lane = lax.broadcasted_iota(jnp.int32, (128,), 0)
    prev = jnp.where(nt == 0, jnp.zeros((128,), jnp.float32), loss_ref[0, 0, :])
    loss_ref[0, 0, :] = prev + jnp.where(lane == 0, part, 0.0)


def _vq_argmin(z, x2, y2, w2):
    return pl.pallas_call(
        _vq_body,
        grid=(B, NT),
        in_specs=[
            pl.BlockSpec((1, D, TT), lambda b, nt: (b, 0, nt)),
            pl.BlockSpec((1, 1, TT), lambda b, nt: (b, 0, nt)),
            pl.BlockSpec((K, 1), lambda b, nt: (0, 0)),
            pl.BlockSpec((K, D), lambda b, nt: (0, 0)),
        ],
        out_specs=[
            pl.BlockSpec((1, 1, T), lambda b, nt: (b, 0, 0)),
            pl.BlockSpec((1, 1, 128), lambda b, nt: (b, 0, 0)),
        ],
        out_shape=[
            jax.ShapeDtypeStruct((B, 1, T), jnp.int32),
            jax.ShapeDtypeStruct((B, 1, 128), jnp.float32),
        ],
        compiler_params=pltpu.CompilerParams(
            dimension_semantics=("parallel", "arbitrary")),
    )(z, x2, y2, w2)


N = B * T
NC, NS = 2, 16
NW = NC * NS
CHUNK = 128
CH = N // (NW * CHUNK)

@functools.cache
def _sc_gather_fn():
    mesh = plsc.VectorSubcoreMesh(core_axis_name="c", subcore_axis_name="s")

    @functools.partial(
        pl.kernel,
        mesh=mesh,
        out_type=jax.ShapeDtypeStruct((N, D), jnp.float32),
        scratch_types=[
            pltpu.VMEM((CH, CHUNK), jnp.int32),
            pltpu.VMEM((CHUNK, D), jnp.float32),
            pltpu.VMEM((CHUNK, D), jnp.float32),
            pltpu.SemaphoreType.DMA,
            pltpu.SemaphoreType.DMA,
        ],
    )
    def _sc_gather(table_hbm, idx_hbm, out_hbm, idx_v, buf0, buf1, sem0, sem1):
        wid = lax.axis_index("s") * NC + lax.axis_index("c")
        base = wid * (CH * CHUNK)
        pltpu.sync_copy(idx_hbm.at[wid], idx_v)
        bufs = (buf0, buf1)
        sems = (sem0, sem1)
        copies = [None, None]
        copies[0] = pltpu.async_copy(table_hbm.at[idx_v.at[0]], bufs[0], sems[0])
        for c in range(CH):
            p = c % 2
            copies[p].wait()
            if c + 1 < CH:
                copies[1 - p] = pltpu.async_copy(
                    table_hbm.at[idx_v.at[c + 1]], bufs[1 - p], sems[1 - p])
            pltpu.sync_copy(bufs[p], out_hbm.at[pl.ds(base + c * CHUNK, CHUNK)])

    return _sc_gather


def kernel(z, W):
    x2 = jnp.sum(z * z, axis=1).reshape(-1)
    y2 = jnp.sum(W ** 2, axis=1)
    idx3, loss_parts = _vq_argmin(
        z, x2.reshape(B, 1, T), y2.reshape(K, 1), 2.0 * W)
    codes = idx3.reshape(1, B, 1, T)
    loss_commit = jnp.sum(loss_parts) / jnp.float32(B * T * D)
    q = _sc_gather_fn()(W, idx3.reshape(NW, CH, CHUNK))
    z_q = q.reshape(B, T, D).transpose(0, 2, 1)
    loss_codebook = jnp.float32(0.0)
    bandwidth = jnp.array([0.075], dtype=jnp.float32)
    return (z_q, codes, loss_commit, loss_codebook, bandwidth)
